# baseline (speedup 1.0000x reference)
import jax
import jax.numpy as jnp
from jax import lax
from jax.experimental import pallas as pl
from jax.experimental.pallas import tpu as pltpu
from jax.experimental.pallas import tpu_sc as plsc

NUM_FIELDS = 26
FIELD_SIZE = 100000
BATCH = 16384
EMBED_DIM = 16
LANES = 16

NC, NS = 2, 16
NW = NC * NS
B_FLAT = BATCH * NUM_FIELDS
PER_W = B_FLAT // NW
PAT = 208
CHUNK = 1664
NCHUNK = PER_W // CHUNK


def _sc_body(x_hbm, table_hbm, out_hbm, idx_v, rows_v, sem):
    wid = lax.axis_index("s") * NC + lax.axis_index("c")
    base = wid * PER_W

    iota = lax.iota(jnp.int32, LANES)
    pat = [((iota + j * LANES) % NUM_FIELDS) * FIELD_SIZE
           for j in range(PAT // LANES)]

    def chunk_body(c, carry):
        off = base + c * CHUNK
        pltpu.sync_copy(x_hbm.at[pl.ds(off, CHUNK)], idx_v)

        def blk_body(b, carry2):
            for j in range(PAT // LANES):
                sl = pl.ds(b * PAT + j * LANES, LANES)
                idx_v[sl] = idx_v[sl] + pat[j]
            return carry2

        lax.fori_loop(0, CHUNK // PAT, blk_body, 0, unroll=False)
        pltpu.async_copy(table_hbm.at[idx_v], rows_v, sem).wait()
        pltpu.sync_copy(rows_v, out_hbm.at[pl.ds(off, CHUNK)])
        return carry

    lax.fori_loop(0, NCHUNK, chunk_body, 0, unroll=False)


def kernel(x, table):
    x_flat = x.reshape(B_FLAT)
    out = pl.kernel(
        _sc_body,
        out_type=jax.ShapeDtypeStruct((B_FLAT, EMBED_DIM), jnp.float32),
        mesh=plsc.VectorSubcoreMesh(
            core_axis_name="c", subcore_axis_name="s",
            num_cores=NC, num_subcores=NS),
        scratch_types=[
            pltpu.VMEM((CHUNK,), jnp.int32),
            pltpu.VMEM((CHUNK, EMBED_DIM), jnp.float32),
            pltpu.SemaphoreType.DMA,
        ],
        compiler_params=pltpu.CompilerParams(use_tc_tiling_on_sc=False),
    )(x_flat, table)
    return out.reshape(BATCH, NUM_FIELDS, EMBED_DIM)


# speedup vs baseline: 1.1927x; 1.1927x over previous
import jax
import jax.numpy as jnp
from jax import lax
from jax.experimental import pallas as pl
from jax.experimental.pallas import tpu as pltpu
from jax.experimental.pallas import tpu_sc as plsc

NUM_FIELDS = 26
FIELD_SIZE = 100000
BATCH = 16384
EMBED_DIM = 16
LANES = 16
NROW = FIELD_SIZE * NUM_FIELDS
B_FLAT = BATCH * NUM_FIELDS

NC, NS = 2, 16
NW = NC * NS
BLK = 512
CT = BLK // 128
NTASK = NUM_FIELDS * (BATCH // BLK)
TASKS_PER_W = NTASK // NW


def _sc_body(idx_hbm, table_hbm, out_hbm, idx_v, sub_v, rows_v, buf_v, sem):
    wid = lax.axis_index("s") * NC + lax.axis_index("c")
    iota = lax.iota(jnp.int32, LANES)

    def task_body(k, carry):
        t = k * NW + wid
        f = t // (BATCH // BLK)
        b0 = (t % (BATCH // BLK)) * BLK
        pltpu.sync_copy(idx_hbm.at[pl.ds(f * BATCH + b0, BLK)], idx_v)
        off = f * FIELD_SIZE

        def add_body(i, carry2):
            sl = pl.ds(i * LANES, LANES)
            r = idx_v[sl] + off
            sub_v[sl] = (r & 7) << 4
            idx_v[sl] = r >> 3
            return carry2

        lax.fori_loop(0, BLK // LANES, add_body, 0, unroll=False)
        pltpu.async_copy(table_hbm.at[idx_v], rows_v, sem).wait()

        def grp_body(g, carry2):
            g16 = g * LANES
            row_idx = g16 + iota
            sub = sub_v[pl.ds(g16, LANES)]
            ct = g16 // 128
            bl = g16 % 128
            for d in range(EMBED_DIM):
                v = plsc.load_gather(rows_v, [row_idx, sub + d])
                buf_v[d // 8, ct, d % 8, pl.ds(bl, LANES)] = v
            return carry2

        lax.fori_loop(0, BLK // LANES, grp_body, 0, unroll=False)
        pltpu.sync_copy(
            buf_v, out_hbm.at[f, :, pl.ds(b0 // 128, CT), :, :])
        return carry

    lax.fori_loop(0, TASKS_PER_W, task_body, 0, unroll=False)


def kernel(x, table):
    idx_flat = x.T.reshape(B_FLAT)
    table_v = table.reshape(NROW * EMBED_DIM // 128, 128)
    out5 = pl.kernel(
        _sc_body,
        out_type=jax.ShapeDtypeStruct((NUM_FIELDS, 2, BATCH // 128, 8, 128),
                                      jnp.float32),
        mesh=plsc.VectorSubcoreMesh(
            core_axis_name="c", subcore_axis_name="s",
            num_cores=NC, num_subcores=NS),
        scratch_types=[
            pltpu.VMEM((BLK,), jnp.int32),
            pltpu.VMEM((BLK,), jnp.int32),
            pltpu.VMEM((BLK, 128), jnp.float32),
            pltpu.VMEM((2, CT, 8, 128), jnp.float32),
            pltpu.SemaphoreType.DMA,
        ],
        compiler_params=pltpu.CompilerParams(
            use_tc_tiling_on_sc=False, needs_layout_passes=False),
    )(idx_flat, table_v)
    out = out5.transpose(2, 4, 0, 1, 3).reshape(BATCH, NUM_FIELDS, EMBED_DIM)
    return out


# speedup vs baseline: 1.2611x; 1.0574x over previous
import jax
import jax.numpy as jnp
from jax import lax
from jax.experimental import pallas as pl
from jax.experimental.pallas import tpu as pltpu
from jax.experimental.pallas import tpu_sc as plsc

NUM_FIELDS = 26
FIELD_SIZE = 100000
BATCH = 16384
EMBED_DIM = 16
LANES = 16
NROW = FIELD_SIZE * NUM_FIELDS
B_FLAT = BATCH * NUM_FIELDS

NC, NS = 2, 16
NW = NC * NS
BLK = 512
CT = BLK // 128
NTASK = NUM_FIELDS * (BATCH // BLK)
TASKS_PER_W = NTASK // NW


def _sc_body(idx_hbm, table_hbm, out_hbm, idx_v, rows_v, buf_v, sem):
    wid = lax.axis_index("s") * NC + lax.axis_index("c")
    iota = lax.iota(jnp.int32, LANES)

    def task_body(k, carry):
        t = k * NW + wid
        f = t // (BATCH // BLK)
        b0 = (t % (BATCH // BLK)) * BLK
        pltpu.sync_copy(idx_hbm.at[pl.ds(f * BATCH + b0, BLK)], idx_v)
        off = f * FIELD_SIZE

        def add_body(i, carry2):
            sl = pl.ds(i * LANES, LANES)
            idx_v[sl] = idx_v[sl] + off
            return carry2

        lax.fori_loop(0, BLK // LANES, add_body, 0, unroll=False)
        pltpu.async_copy(table_hbm.at[idx_v], rows_v, sem).wait()

        def grp_body(g, carry2):
            g16 = g * LANES
            row_idx = g16 + iota
            ct = g16 // 128
            bl = g16 % 128
            for d in range(EMBED_DIM):
                v = plsc.load_gather(rows_v, [row_idx, iota * 0 + d])
                buf_v[d // 8, ct, d % 8, pl.ds(bl, LANES)] = v
            return carry2

        lax.fori_loop(0, BLK // LANES, grp_body, 0, unroll=False)
        pltpu.sync_copy(
            buf_v, out_hbm.at[f, :, pl.ds(b0 // 128, CT), :, :])
        return carry

    lax.fori_loop(0, TASKS_PER_W, task_body, 0, unroll=False)


def kernel(x, table):
    idx_flat = x.T.reshape(B_FLAT)
    out5 = pl.kernel(
        _sc_body,
        out_type=jax.ShapeDtypeStruct((NUM_FIELDS, 2, BATCH // 128, 8, 128),
                                      jnp.float32),
        mesh=plsc.VectorSubcoreMesh(
            core_axis_name="c", subcore_axis_name="s",
            num_cores=NC, num_subcores=NS),
        scratch_types=[
            pltpu.VMEM((BLK,), jnp.int32),
            pltpu.VMEM((BLK, EMBED_DIM), jnp.float32),
            pltpu.VMEM((2, CT, 8, 128), jnp.float32),
            pltpu.SemaphoreType.DMA,
        ],
        compiler_params=pltpu.CompilerParams(
            use_tc_tiling_on_sc=False, needs_layout_passes=False),
    )(idx_flat, table)
    out = out5.transpose(2, 4, 0, 1, 3).reshape(BATCH, NUM_FIELDS, EMBED_DIM)
    return out
